# baseline (speedup 1.0000x reference)
import jax
import jax.numpy as jnp
from jax import lax
from jax.experimental import pallas as pl
from jax.experimental.pallas import tpu as pltpu

ROWS = 100000
COLS = 512
N_TOTAL = float(ROWS * COLS)

SMALL = 400
N_SMALL = 5
BIG = 2000
N_BIG = 49
N_CHUNKS = N_SMALL + N_BIG


def _row0(c):
    if isinstance(c, int):
        return c * SMALL if c < N_SMALL else (c - 4) * BIG
    return (c - 4) * BIG


def _body(o_hbm, t_hbm, m_hbm, loss_ref, bo, bt, bm, sems, acc_ref):
    def issue(c, nrows, slot):
        r0 = _row0(c) if isinstance(c, int) else _row0(c)
        pltpu.make_async_copy(
            o_hbm.at[pl.ds(r0, nrows)], bo.at[slot, pl.ds(0, nrows)],
            sems.at[slot, 0],
        ).start()
        pltpu.make_async_copy(
            t_hbm.at[pl.ds(r0, nrows)], bt.at[slot, pl.ds(0, nrows)],
            sems.at[slot, 1],
        ).start()
        pltpu.make_async_copy(
            m_hbm.at[pl.ds(r0, nrows)], bm.at[slot, pl.ds(0, nrows)],
            sems.at[slot, 2],
        ).start()

    def wait(c, nrows, slot):
        r0 = _row0(c)
        pltpu.make_async_copy(
            o_hbm.at[pl.ds(r0, nrows)], bo.at[slot, pl.ds(0, nrows)],
            sems.at[slot, 0],
        ).wait()
        pltpu.make_async_copy(
            t_hbm.at[pl.ds(r0, nrows)], bt.at[slot, pl.ds(0, nrows)],
            sems.at[slot, 1],
        ).wait()
        pltpu.make_async_copy(
            m_hbm.at[pl.ds(r0, nrows)], bm.at[slot, pl.ds(0, nrows)],
            sems.at[slot, 2],
        ).wait()

    def accumulate(nrows, slot):
        o = bo[slot, pl.ds(0, nrows), :]
        t = bt[slot, pl.ds(0, nrows), :]
        m = bm[slot, pl.ds(0, nrows), :].astype(jnp.float32)
        d = o - t
        d2 = d * d
        acc_ref[0] += jnp.sum(d2 * m)
        acc_ref[1] += jnp.sum(d2)
        acc_ref[2] += jnp.sum(m)

    acc_ref[0] = 0.0
    acc_ref[1] = 0.0
    acc_ref[2] = 0.0

    issue(0, SMALL, 0)
    issue(1, SMALL, 1)

    for c in range(N_SMALL):
        slot = c % 2
        wait(c, SMALL, slot)
        nxt = c + 2
        if nxt < N_SMALL:
            issue(nxt, SMALL, nxt % 2)
        elif nxt < N_CHUNKS:
            issue(nxt, BIG, nxt % 2)
        accumulate(SMALL, slot)

    def loop_body(j, _):
        c1 = 5 + 2 * j
        wait(c1, BIG, 1)
        issue(c1 + 2, BIG, 1)
        accumulate(BIG, 1)

        c2 = 6 + 2 * j
        wait(c2, BIG, 0)

        @pl.when(j < 23)
        def _():
            issue(c2 + 2, BIG, 0)

        accumulate(BIG, 0)
        return 0

    lax.fori_loop(0, 24, loop_body, 0)

    wait(N_CHUNKS - 1, BIG, 1)
    accumulate(BIG, 1)

    s_m = acc_ref[0]
    s_tot = acc_ref[1]
    c = acc_ref[2]
    loss_ref[0] = s_m / jnp.maximum(c, 1.0) + (s_tot - s_m) / jnp.maximum(
        N_TOTAL - c, 1.0
    )


def kernel(output, target, mask):
    loss = pl.pallas_call(
        _body,
        in_specs=[
            pl.BlockSpec(memory_space=pl.ANY),
            pl.BlockSpec(memory_space=pl.ANY),
            pl.BlockSpec(memory_space=pl.ANY),
        ],
        out_specs=pl.BlockSpec(memory_space=pltpu.SMEM),
        out_shape=jax.ShapeDtypeStruct((1,), jnp.float32),
        scratch_shapes=[
            pltpu.VMEM((2, BIG, COLS), jnp.float32),
            pltpu.VMEM((2, BIG, COLS), jnp.float32),
            pltpu.VMEM((2, BIG, COLS), jnp.int32),
            pltpu.SemaphoreType.DMA((2, 3)),
            pltpu.SMEM((3,), jnp.float32),
        ],
    )(output, target, mask)
    return loss[0]


# speedup vs baseline: 1.0011x; 1.0011x over previous
import jax
import jax.numpy as jnp
from jax import lax
from jax.experimental import pallas as pl
from jax.experimental.pallas import tpu as pltpu

ROWS = 100000
COLS = 512
N_TOTAL = float(ROWS * COLS)

SMALL = 400
BIG = 2000
N_HEAD = 5
N_BIG = 48
N_TAIL = 5
N_CHUNKS = N_HEAD + N_BIG + N_TAIL


def _row0(c):
    if isinstance(c, int):
        if c < N_HEAD:
            return c * SMALL
        if c < N_HEAD + N_BIG:
            return 2000 + (c - N_HEAD) * BIG
        return 98000 + (c - N_HEAD - N_BIG) * SMALL
    return (c - 4) * BIG


def _nrows(c):
    return BIG if N_HEAD <= c < N_HEAD + N_BIG else SMALL


def _body(o_hbm, t_hbm, m_hbm, loss_ref, bo, bt, bm, sems, acc_ref):
    def copies(c, nrows, slot):
        r0 = _row0(c)
        return [
            pltpu.make_async_copy(
                o_hbm.at[pl.ds(r0, nrows)], bo.at[slot, pl.ds(0, nrows)],
                sems.at[slot, 0],
            ),
            pltpu.make_async_copy(
                t_hbm.at[pl.ds(r0, nrows)], bt.at[slot, pl.ds(0, nrows)],
                sems.at[slot, 1],
            ),
            pltpu.make_async_copy(
                m_hbm.at[pl.ds(r0, nrows)], bm.at[slot, pl.ds(0, nrows)],
                sems.at[slot, 2],
            ),
        ]

    def issue(c, nrows, slot):
        for cp in copies(c, nrows, slot):
            cp.start()

    def wait(c, nrows, slot):
        for cp in copies(c, nrows, slot):
            cp.wait()

    def accumulate(nrows, slot):
        o = bo[slot, pl.ds(0, nrows), :]
        t = bt[slot, pl.ds(0, nrows), :]
        m = bm[slot, pl.ds(0, nrows), :].astype(jnp.float32)
        d = o - t
        d2 = d * d
        acc_ref[0] += jnp.sum(d2 * m)
        acc_ref[1] += jnp.sum(d2)
        acc_ref[2] += jnp.sum(m)

    acc_ref[0] = 0.0
    acc_ref[1] = 0.0
    acc_ref[2] = 0.0

    issue(0, SMALL, 0)
    issue(1, SMALL, 1)
    issue(2, SMALL, 2)

    for c in range(N_HEAD):
        wait(c, SMALL, c % 3)
        accumulate(SMALL, c % 3)
        issue(c + 3, _nrows(c + 3), (c + 3) % 3)

    def loop_body(j, _):
        for k, slot in ((0, 2), (1, 0), (2, 1)):
            c = 5 + 3 * j + k
            wait(c, BIG, slot)
            accumulate(BIG, slot)
            issue(c + 3, BIG, slot)
        return 0

    lax.fori_loop(0, 15, loop_body, 0)

    for c in (50, 51, 52):
        wait(c, BIG, c % 3)
        accumulate(BIG, c % 3)
        issue(c + 3, SMALL, (c + 3) % 3)

    for c in range(53, N_CHUNKS):
        wait(c, SMALL, c % 3)
        accumulate(SMALL, c % 3)
        if c + 3 < N_CHUNKS:
            issue(c + 3, SMALL, (c + 3) % 3)

    s_m = acc_ref[0]
    s_tot = acc_ref[1]
    cnt = acc_ref[2]
    loss_ref[0] = s_m / jnp.maximum(cnt, 1.0) + (s_tot - s_m) / jnp.maximum(
        N_TOTAL - cnt, 1.0
    )


def kernel(output, target, mask):
    loss = pl.pallas_call(
        _body,
        in_specs=[
            pl.BlockSpec(memory_space=pl.ANY),
            pl.BlockSpec(memory_space=pl.ANY),
            pl.BlockSpec(memory_space=pl.ANY),
        ],
        out_specs=pl.BlockSpec(memory_space=pltpu.SMEM),
        out_shape=jax.ShapeDtypeStruct((1,), jnp.float32),
        scratch_shapes=[
            pltpu.VMEM((3, BIG, COLS), jnp.float32),
            pltpu.VMEM((3, BIG, COLS), jnp.float32),
            pltpu.VMEM((3, BIG, COLS), jnp.int32),
            pltpu.SemaphoreType.DMA((3, 3)),
            pltpu.SMEM((3,), jnp.float32),
        ],
    )(output, target, mask)
    return loss[0]
